# baseline (speedup 1.0000x reference)
import jax
import jax.numpy as jnp
from jax import lax
from jax.experimental import pallas as pl
from jax.experimental.pallas import tpu as pltpu
from jax.experimental.pallas import tpu_sc as plsc

_NBINS = 128
_OUT = 16
_N = 1048576
_NFACE = _NBINS * _NBINS
_NCELL = _NFACE * _NBINS

_NW = 32
_PTS_PER_W = _N // _NW
_CHUNK = 1024
_NCH = _PTS_PER_W // _CHUNK
_GID = 128
_NG = _CHUNK // _GID
_IDXR_W = _PTS_PER_W // 16
_HALF = _N * 8


def _idx_body(x_ref, y_ref, z_ref, o_ref):
    fmax = jnp.float32(_NBINS - 1)
    fzero = jnp.float32(0.0)
    ix = jnp.minimum(jnp.maximum(x_ref[...] * _NBINS, fzero), fmax).astype(jnp.int32)
    iy = jnp.minimum(jnp.maximum(y_ref[...] * _NBINS, fzero), fmax).astype(jnp.int32)
    iz = jnp.minimum(jnp.maximum(z_ref[...] * _NBINS, fzero), fmax).astype(jnp.int32)
    o_ref[...] = (ix * 16 + (iy >> 3)) * 1024 + iz * 8 + (iy & 7)


def _idx_tc(xs, ys, zs):
    nrow = _N // 128
    br = 512
    spec = pl.BlockSpec((br, 128), lambda i: (i, 0))
    return pl.pallas_call(
        _idx_body,
        grid=(nrow // br,),
        in_specs=[spec, spec, spec],
        out_specs=pl.BlockSpec((br, 128), lambda i: (i, 0)),
        out_shape=jax.ShapeDtypeStruct((nrow, 128), jnp.int32),
    )(xs, ys, zs)


_BF8 = 16
_NF8 = _NFACE // 8


def _tab_body(g_ref, o_ref):
    o_ref[...] = jnp.swapaxes(g_ref[...] * jnp.float32(5.0), 1, 2)


def _tab_tc(gv):
    return pl.pallas_call(
        _tab_body,
        grid=(_NF8 // _BF8,),
        in_specs=[pl.BlockSpec((_BF8, 128, 128), lambda i: (i, 0, 0))],
        out_specs=pl.BlockSpec((_BF8, 128, 128), lambda i: (i, 0, 0)),
        out_shape=jax.ShapeDtypeStruct((_NF8, 128, 128), jnp.float32),
    )(gv)


def _gather_body(idx_hbm, tab_hbm, out_hbm, idx_v, idxp0, idxp1, buf0, buf1,
                 sg0, sg1, so0, so1):
    wid = lax.axis_index("s") * 2 + lax.axis_index("c")
    row0 = wid * _IDXR_W
    pltpu.sync_copy(idx_hbm.at[pl.ds(row0, _IDXR_W)], idx_v)

    bufs = (buf0, buf1)
    idxp = (idxp0, idxp1)
    sg = (sg0, sg1)
    so = (so0, so1)
    lanes16 = lax.iota(jnp.int32, 16)

    def permute(c):
        dst = idxp[c % 2]
        for t in range(8):

            @pl.loop(0, 8, unroll=8)
            def _pm(c8):
                vals = idx_v[(c * 8 + c8) * 8 + t]
                plsc.store_scatter(dst, [lanes16 * 8 + (t * 128 + c8)], vals)

    def fire(c):
        buf = bufs[c % 2]
        return [
            pltpu.async_copy(
                tab_hbm.at[idxp[c % 2].at[pl.ds(g * _GID, _GID)]],
                buf.at[pl.ds(g * _GID, _GID)],
                sg[c % 2],
            )
            for g in range(_NG)
        ]

    gd = [None, None]
    wb = [None, None]
    permute(0)
    gd[0] = fire(0)
    permute(1)
    for c in range(_NCH):
        if c + 1 < _NCH:
            if c >= 1:
                wb[(c + 1) % 2].wait()
            gd[(c + 1) % 2] = fire(c + 1)
        for d in gd[c % 2]:
            d.wait()
        if c + 2 < _NCH:
            permute(c + 2)
        wb[c % 2] = pltpu.async_copy(
            bufs[c % 2], out_hbm.at[wid * _NCH + c], so[c % 2]
        )
    for c in (_NCH - 2, _NCH - 1):
        wb[c % 2].wait()


_BA = 16
_NCHT = _N // _CHUNK


def _outt_body(g_ref, o_ref):
    t = jnp.swapaxes(g_ref[...], 1, 2)
    v = t.reshape(_BA, 16, 8, 128)
    o_ref[0] = jnp.concatenate(
        [v[:, 2 * g] for g in range(8)], axis=1).reshape(_BA * 64, 128)
    o_ref[1] = jnp.concatenate(
        [v[:, 2 * g + 1] for g in range(8)], axis=1).reshape(_BA * 64, 128)


def _outt_tc(y):
    return pl.pallas_call(
        _outt_body,
        grid=(_NCHT // _BA,),
        in_specs=[pl.BlockSpec((_BA, 128, 128), lambda i: (i, 0, 0))],
        out_specs=pl.BlockSpec((2, _BA * 64, 128), lambda i: (0, i, 0)),
        out_shape=jax.ShapeDtypeStruct((2, _NCHT * 64, 128), jnp.float32),
    )(y)


@jax.jit
def _run(inputs, grid):
    xs = inputs[:, 0].reshape(_N // 128, 128)
    ys = inputs[:, 1].reshape(_N // 128, 128)
    zs = inputs[:, 2].reshape(_N // 128, 128)
    idx = _idx_tc(xs, ys, zs).reshape(_N // 16, 16)

    gv = jnp.transpose(grid, (0, 1, 3, 2)).reshape(_NF8, 128, 128)
    tab = _tab_tc(gv).reshape(_NCELL, _OUT)

    mesh = plsc.VectorSubcoreMesh(core_axis_name="c", subcore_axis_name="s")
    y = pl.kernel(
        _gather_body,
        out_type=jax.ShapeDtypeStruct((_NCHT, _CHUNK, _OUT), jnp.float32),
        mesh=mesh,
        scratch_types=[
            pltpu.VMEM((_IDXR_W, 16), jnp.int32),
            pltpu.VMEM((_CHUNK,), jnp.int32),
            pltpu.VMEM((_CHUNK,), jnp.int32),
            pltpu.VMEM((_CHUNK, _OUT), jnp.float32),
            pltpu.VMEM((_CHUNK, _OUT), jnp.float32),
            pltpu.SemaphoreType.DMA,
            pltpu.SemaphoreType.DMA,
            pltpu.SemaphoreType.DMA,
            pltpu.SemaphoreType.DMA,
        ],
        compiler_params=pltpu.CompilerParams(
            needs_layout_passes=False, use_tc_tiling_on_sc=False
        ),
    )(idx, tab)
    yt = _outt_tc(y.reshape(_NCHT, 128, 128))
    return yt.reshape(2, _N // 128, 8, 128).transpose(1, 3, 0, 2).reshape(_N, _OUT)


def kernel(inputs, grid):
    return _run(inputs, grid)


# speedup vs baseline: 1.8434x; 1.8434x over previous
import jax
import jax.numpy as jnp
from jax import lax
from jax.experimental import pallas as pl
from jax.experimental.pallas import tpu as pltpu
from jax.experimental.pallas import tpu_sc as plsc

_NBINS = 128
_OUT = 16
_N = 1048576
_NFACE = _NBINS * _NBINS
_NCELL = _NFACE * _NBINS

_NW = 32
_PTS_PER_W = _N // _NW
_CHUNK = 1024
_NCH = _PTS_PER_W // _CHUNK
_GID = 128
_NG = _CHUNK // _GID
_CB = _CHUNK * _OUT
_BAND = _CB // 2


def _idx_body(x_ref, y_ref, z_ref, o_ref):
    fmax = jnp.float32(_NBINS - 1)
    fzero = jnp.float32(0.0)
    ix = jnp.minimum(jnp.maximum(x_ref[...] * _NBINS, fzero), fmax).astype(jnp.int32)
    iy = jnp.minimum(jnp.maximum(y_ref[...] * _NBINS, fzero), fmax).astype(jnp.int32)
    iz = jnp.minimum(jnp.maximum(z_ref[...] * _NBINS, fzero), fmax).astype(jnp.int32)
    o_ref[...] = (ix * 16 + (iy >> 3)) * 1024 + iz * 8 + (iy & 7)


def _idx_tc(xs, ys, zs):
    nrow = _N // 128
    br = 512
    spec = pl.BlockSpec((br, 128), lambda i: (i, 0))
    return pl.pallas_call(
        _idx_body,
        grid=(nrow // br,),
        in_specs=[spec, spec, spec],
        out_specs=pl.BlockSpec((br, 128), lambda i: (i, 0)),
        out_shape=jax.ShapeDtypeStruct((nrow, 128), jnp.int32),
    )(xs, ys, zs)


_BF8 = 16
_NF8 = _NFACE // 8


def _tab_body(g_ref, o_ref):
    o_ref[...] = jnp.swapaxes(g_ref[...] * jnp.float32(5.0), 1, 2)


def _tab_tc(gv):
    return pl.pallas_call(
        _tab_body,
        grid=(_NF8 // _BF8,),
        in_specs=[pl.BlockSpec((_BF8, 128, 128), lambda i: (i, 0, 0))],
        out_specs=pl.BlockSpec((_BF8, 128, 128), lambda i: (i, 0, 0)),
        out_shape=jax.ShapeDtypeStruct((_NF8, 128, 128), jnp.float32),
    )(gv)


def _gather_body(idx_hbm, tab_hbm, out_hbm, idx_v, buf0, buf1, ob0, ob1,
                 sg0, sg1, so0, so1):
    wid = lax.axis_index("s") * 2 + lax.axis_index("c")
    pltpu.sync_copy(idx_hbm.at[pl.ds(wid * _PTS_PER_W, _PTS_PER_W)], idx_v)

    bufs = (buf0, buf1)
    obs = (ob0, ob1)
    sg = (sg0, sg1)
    so = (so0, so1)
    lanes16 = lax.iota(jnp.int32, 16)
    offbase = (lanes16 >> 3) * _BAND + (lanes16 & 7) * 128

    def fire(c):
        buf = bufs[c % 2]
        return [
            pltpu.async_copy(
                tab_hbm.at[idx_v.at[pl.ds(c * _CHUNK + g * _GID, _GID)]],
                buf.at[pl.ds(g * _GID, _GID)],
                sg[c % 2],
            )
            for g in range(_NG)
        ]

    def xpose(c):
        buf = bufs[c % 2]
        ob = obs[c % 2]

        @pl.loop(0, _CHUNK, unroll=8)
        def _tp(q):
            vals = buf[q]
            off = (q >> 7) * 1024 + (q & 127)
            plsc.store_scatter(ob, [offbase + off], vals)

    gd = [None, None]
    wb = [None, None]
    gd[0] = fire(0)
    for c in range(_NCH):
        if c + 1 < _NCH:
            gd[(c + 1) % 2] = fire(c + 1)
        for d in gd[c % 2]:
            d.wait()
        if c >= 2:
            for w in wb[c % 2]:
                w.wait()
        xpose(c)
        ob = obs[c % 2]
        wb[c % 2] = [
            pltpu.async_copy(
                ob.at[pl.ds(b * _BAND, _BAND)],
                out_hbm.at[b, wid * _NCH + c],
                so[c % 2],
            )
            for b in range(2)
        ]
    for c in (_NCH - 2, _NCH - 1):
        for w in wb[c % 2]:
            w.wait()


@jax.jit
def _run(inputs, grid):
    xs = inputs[:, 0].reshape(_N // 128, 128)
    ys = inputs[:, 1].reshape(_N // 128, 128)
    zs = inputs[:, 2].reshape(_N // 128, 128)
    idx = _idx_tc(xs, ys, zs).reshape(_N)

    gv = jnp.transpose(grid, (0, 1, 3, 2)).reshape(_NF8, 128, 128)
    tab = _tab_tc(gv).reshape(_NCELL, _OUT)

    mesh = plsc.VectorSubcoreMesh(core_axis_name="c", subcore_axis_name="s")
    y = pl.kernel(
        _gather_body,
        out_type=jax.ShapeDtypeStruct((2, _N // _CHUNK, _BAND), jnp.float32),
        mesh=mesh,
        scratch_types=[
            pltpu.VMEM((_PTS_PER_W,), jnp.int32),
            pltpu.VMEM((_CHUNK, _OUT), jnp.float32),
            pltpu.VMEM((_CHUNK, _OUT), jnp.float32),
            pltpu.VMEM((_CB,), jnp.float32),
            pltpu.VMEM((_CB,), jnp.float32),
            pltpu.SemaphoreType.DMA,
            pltpu.SemaphoreType.DMA,
            pltpu.SemaphoreType.DMA,
            pltpu.SemaphoreType.DMA,
        ],
        compiler_params=pltpu.CompilerParams(
            needs_layout_passes=False, use_tc_tiling_on_sc=False
        ),
    )(idx, tab)
    return (
        y.reshape(2, _N // 128, 8, 128).transpose(1, 3, 0, 2).reshape(_N, _OUT)
    )


def kernel(inputs, grid):
    return _run(inputs, grid)
